# baseline (speedup 1.0000x reference)
import functools

import jax
import jax.numpy as jnp
from jax import lax
from jax.experimental import pallas as pl
from jax.experimental.pallas import tpu as pltpu
from jax.experimental.pallas import tpu_sc as plsc

_F32 = jnp.float32

_N = 10000
_E = 160000
_H = 4
_D = 64
_HD = _H * _D
_B = 64
_EMB = 64
_DM = 128

_NC, _NS = 2, 16
_NW = _NC * _NS
_EW = _E // _NW
_CG = 40
_NCH = _EW // _CG
_ET = _E // _NS
_CGV = 40
_CHTV = _ET // _CGV
_NP = 10240
_NROW = _NP // _NS


def _mesh():
    return plsc.VectorSubcoreMesh(
        core_axis_name="c", subcore_axis_name="s",
        num_cores=_NC, num_subcores=_NS)


def _proj(x, p):
    n, c = x.shape
    rb = 2000

    def body(x_ref, wq, bq, wk, bk, wv, bv, ws, bs, q_o, k_o, vt_o, xr_o):
        xb = x_ref[...]
        q_o[...] = jnp.dot(xb, wq[...], preferred_element_type=_F32) + bq[...]
        k_o[...] = jnp.dot(xb, wk[...], preferred_element_type=_F32) + bk[...]
        v = jnp.dot(xb, wv[...], preferred_element_type=_F32) + bv[...]
        vt_o[0] = v[:, 0:_DM]
        vt_o[1] = v[:, _DM:2 * _DM]
        xr_o[...] = jnp.dot(xb, ws[...], preferred_element_type=_F32) + bs[...]

    def full(shape):
        return pl.BlockSpec(shape, lambda i: (0, 0))

    row = lambda w: pl.BlockSpec((rb, w), lambda i: (i, 0))
    return pl.pallas_call(
        body, grid=(n // rb,),
        in_specs=[row(c),
                  full((c, _HD)), full((1, _HD)),
                  full((c, _HD)), full((1, _HD)),
                  full((c, _HD)), full((1, _HD)),
                  full((c, _D)), full((1, _D))],
        out_specs=[row(_HD), row(_HD),
                   pl.BlockSpec((_NC, rb, _DM), lambda i: (0, i, 0)), row(_D)],
        out_shape=[jax.ShapeDtypeStruct((n, _HD), _F32)] * 2
                 + [jax.ShapeDtypeStruct((_NC, n, _DM), _F32),
                    jax.ShapeDtypeStruct((n, _D), _F32)],
    )(x, p['Wq'], p['bq'].reshape(1, -1), p['Wk'], p['bk'].reshape(1, -1),
      p['Wv'], p['bv'].reshape(1, -1), p['Wskip'], p['bskip'].reshape(1, -1))


def _gather_alpha(q_tab, k_tab, dst1d, src1d):
    @functools.partial(
        pl.kernel,
        out_type=jax.ShapeDtypeStruct((_E, 4 * 16), _F32),
        mesh=_mesh(),
        scratch_types=[
            pltpu.VMEM((_EW,), jnp.int32),
            pltpu.VMEM((_EW,), jnp.int32),
            pltpu.VMEM((2, _CG, _HD), _F32),
            pltpu.VMEM((2, _CG, _HD), _F32),
            pltpu.VMEM((2, _CG, 4 * 16), _F32),
            pltpu.SemaphoreType.DMA,
            pltpu.SemaphoreType.DMA,
            pltpu.SemaphoreType.DMA,
            pltpu.SemaphoreType.DMA,
        ],
    )
    def k(q_hbm, k_hbm, d_hbm, s_hbm, alpha_hbm,
          dv, sv, qv, kv, av, gs0, gs1, ss0, ss1):
        wid = lax.axis_index("s") * _NC + lax.axis_index("c")
        ebase = wid * _EW
        pltpu.sync_copy(d_hbm.at[pl.ds(ebase, _EW)], dv)
        pltpu.sync_copy(s_hbm.at[pl.ds(ebase, _EW)], sv)
        gsem = (gs0, gs1)
        ssem = (ss0, ss1)

        def g_desc(j, b):
            return (pltpu.make_async_copy(
                        q_hbm.at[dv.at[pl.ds(j * _CG, _CG)]], qv.at[b], gsem[b]),
                    pltpu.make_async_copy(
                        k_hbm.at[sv.at[pl.ds(j * _CG, _CG)]], kv.at[b], gsem[b]))

        def s_desc(j, b):
            off = ebase + j * _CG
            return (pltpu.make_async_copy(
                av.at[b], alpha_hbm.at[pl.ds(off, _CG), :], ssem[b]),)

        def fire(ds):
            for d in ds:
                d.start()

        def drain(ds):
            for d in ds:
                d.wait()

        def compute(b):
            def edge_body(eo, carry):
                for u in range(4):
                    e = eo * 4 + u
                    for h in range(_H):
                        acc = None
                        for i in range(4):
                            sl = pl.ds(h * _D + i * 16, 16)
                            p = qv[b, e, sl] * kv[b, e, sl]
                            acc = p if acc is None else acc + p
                        av[b, e, pl.ds(h * 16, 16)] = acc * 0.125
                return carry

            lax.fori_loop(0, _CG // 4, edge_body, 0)

        fire(g_desc(0, 0))
        fire(g_desc(1, 1))
        drain(g_desc(0, 0))
        compute(0)
        fire(s_desc(0, 0))

        def pair(g, carry):
            jo = 2 * g + 1
            drain(s_desc(jo - 1, 0))
            fire(g_desc(jo + 1, 0))
            drain(g_desc(jo, 1))
            compute(1)
            fire(s_desc(jo, 1))
            je = jo + 1
            drain(s_desc(je - 1, 1))
            fire(g_desc(je + 1, 1))
            drain(g_desc(je, 0))
            compute(0)
            fire(s_desc(je, 0))
            return carry

        lax.fori_loop(0, (_NCH - 3) // 2, pair, 0)
        jo = _NCH - 2
        drain(s_desc(jo - 1, 0))
        fire(g_desc(jo + 1, 0))
        drain(g_desc(jo, 1))
        compute(1)
        fire(s_desc(jo, 1))
        je = _NCH - 1
        drain(s_desc(je - 1, 1))
        drain(g_desc(je, 0))
        compute(0)
        fire(s_desc(je, 0))
        drain(s_desc(je, 0))

    return k(q_tab, k_tab, dst1d, src1d)


def _alpha_red(alpha64):
    rb = 4000

    def body(a_ref, alpha_ref, mx_ref):
        x = a_ref[...]
        parts = [jnp.sum(x[:, h * 16:(h + 1) * 16], axis=1, keepdims=True)
                 for h in range(_H)]
        a = jnp.concatenate(parts, axis=1)
        alpha_ref[...] = a
        bm = jnp.max(a, axis=0, keepdims=True)
        i = pl.program_id(0)

        @pl.when(i == 0)
        def _():
            mx_ref[...] = bm

        @pl.when(i != 0)
        def _():
            mx_ref[...] = jnp.maximum(mx_ref[...], bm)

    return pl.pallas_call(
        body, grid=(_E // rb,),
        in_specs=[pl.BlockSpec((rb, 4 * 16), lambda i: (i, 0))],
        out_specs=[pl.BlockSpec((rb, _H), lambda i: (i, 0)),
                   pl.BlockSpec((1, _H), lambda i: (0, 0))],
        out_shape=[jax.ShapeDtypeStruct((_E, _H), _F32),
                   jax.ShapeDtypeStruct((1, _H), _F32)],
    )(alpha64)


def _exrep(alpha, mx):
    rb = 8000

    def body(alpha_ref, mx_ref, er_ref):
        ex = jnp.exp(alpha_ref[...] - mx_ref[...])
        for c in range(_NC):
            e0 = jnp.broadcast_to(ex[:, 2 * c:2 * c + 1], (rb, 16))
            e1 = jnp.broadcast_to(ex[:, 2 * c + 1:2 * c + 2], (rb, 16))
            er_ref[c] = jnp.concatenate([e0, e1], axis=1)

    return pl.pallas_call(
        body, grid=(_E // rb,),
        in_specs=[pl.BlockSpec((rb, _H), lambda i: (i, 0)),
                  pl.BlockSpec((1, _H), lambda i: (0, 0))],
        out_specs=pl.BlockSpec((_NC, rb, 32), lambda i: (0, i, 0)),
        out_shape=jax.ShapeDtypeStruct((_NC, _E, 32), _F32),
    )(alpha, mx)


def _edge_scatter(vt, exrep, dst1d, src2, zrows):
    @functools.partial(
        pl.kernel,
        out_type=[jax.ShapeDtypeStruct((_NC, _NP, _DM), _F32)] * 2,
        mesh=_mesh(),
        scratch_types=[
            pltpu.VMEM((2, _CGV), jnp.int32),
            pltpu.VMEM((2, _CGV), jnp.int32),
            pltpu.VMEM((2, _CGV, _DM), _F32),
            pltpu.VMEM((2, _CGV, 32), _F32),
            pltpu.VMEM_SHARED((_NP, _DM), _F32),
            pltpu.SemaphoreType.DMA,
            pltpu.SemaphoreType.DMA,
            pltpu.SemaphoreType.DMA,
            pltpu.SemaphoreType.DMA,
            pltpu.SemaphoreType.DMA,
            pltpu.SemaphoreType.DMA,
        ],
    )
    def k(vt_hbm, er_hbm, d_hbm, s_hbm, z_hbm, acc_out, den_out,
          didx, sidx, vb, exr, acc_sh, is0, is1, gs0, gs1, cs0, cs1):
        c = lax.axis_index("c")
        s = lax.axis_index("s")
        pltpu.sync_copy(z_hbm, acc_sh.at[pl.ds(s * _NROW, _NROW)])
        isem = (is0, is1)
        gsem = (gs0, gs1)
        csem = (cs0, cs1)
        iota16 = lax.iota(jnp.int32, 16)
        plsc.subcore_barrier()

        def i_desc(j, b, with_src):
            off = s * _ET + j * _CGV
            ds_ = [pltpu.make_async_copy(
                       d_hbm.at[pl.ds(off, _CGV)], didx.at[b], isem[b]),
                   pltpu.make_async_copy(
                       er_hbm.at[c, pl.ds(off, _CGV), :], exr.at[b], isem[b])]
            if with_src:
                ds_.append(pltpu.make_async_copy(
                    s_hbm.at[pl.ds(c * _E + off, _CGV)], sidx.at[b], isem[b]))
            return ds_

        def g_desc(j, b):
            return (pltpu.make_async_copy(
                vt_hbm.at[sidx.at[b]], vb.at[b], gsem[b]),)

        def c_desc(j, b):
            return (pltpu.make_async_copy(
                vb.at[b], acc_sh.at[didx.at[b]], csem[b]),)

        def fire(ds):
            for d in ds:
                d.start()

        def drain(ds):
            for d in ds:
                d.wait()

        def fire_add(j, b):
            pltpu.async_copy(vb.at[b], acc_sh.at[didx.at[b]], csem[b], add=True)

        def v_compute(b):
            def edge_body(eo, carry):
                for u in range(4):
                    e = eo * 4 + u
                    b0 = exr[b, e, pl.ds(0, 16)]
                    b1 = exr[b, e, pl.ds(16, 16)]
                    for i in range(8):
                        sl = pl.ds(i * 16, 16)
                        bb = b0 if i < 4 else b1
                        vb[b, e, sl] = vb[b, e, sl] * bb
                return carry

            lax.fori_loop(0, _CGV // 4, edge_body, 0)

        def v_step(j, b, nb, first, last):
            drain(g_desc(j, b))
            if not first:
                drain(c_desc(j - 1, nb))
            if not last:
                fire(i_desc(j + 1, nb, True))
            v_compute(b)
            if not last:
                drain(i_desc(j + 1, nb, True))
                fire(g_desc(j + 1, nb))
            fire_add(j, b)

        fire(i_desc(0, 0, True))
        drain(i_desc(0, 0, True))
        fire(g_desc(0, 0))
        v_step(0, 0, 1, True, False)

        def v_pair(g, carry):
            jo = 2 * g + 1
            v_step(jo, 1, 0, False, False)
            v_step(jo + 1, 0, 1, False, False)
            return carry

        lax.fori_loop(0, (_CHTV - 2) // 2, v_pair, 0)
        v_step(_CHTV - 1, 1, 0, False, True)
        drain(c_desc(_CHTV - 1, 1))

        plsc.subcore_barrier()
        pltpu.sync_copy(acc_sh.at[pl.ds(s * _NROW, _NROW)],
                        acc_out.at[c, pl.ds(s * _NROW, _NROW), :])
        plsc.subcore_barrier()
        pltpu.sync_copy(z_hbm, acc_sh.at[pl.ds(s * _NROW, _NROW)])
        plsc.subcore_barrier()

        pltpu.sync_copy(z_hbm.at[pl.ds(0, _CGV)], vb.at[0])
        pltpu.sync_copy(z_hbm.at[pl.ds(0, _CGV)], vb.at[1])

        def d_compute(b):
            def edge_body(eo, carry):
                for u in range(4):
                    e = eo * 4 + u
                    v0 = exr[b, e, pl.ds(0, 16)]
                    v1 = exr[b, e, pl.ds(16, 16)]
                    m = jnp.where(iota16 == 2 * c, v0,
                                  jnp.where(iota16 == 2 * c + 1, v1,
                                            jnp.zeros((16,), _F32)))
                    vb[b, e, pl.ds(0, 16)] = m
                return carry

            lax.fori_loop(0, _CGV // 4, edge_body, 0)

        def d_step(j, b, nb, first, last):
            drain(i_desc(j, b, False))
            if not first:
                drain(c_desc(j - 1, nb))
            if not last:
                fire(i_desc(j + 1, nb, False))
            d_compute(b)
            fire_add(j, b)

        fire(i_desc(0, 0, False))
        d_step(0, 0, 1, True, False)

        def d_pair(g, carry):
            jo = 2 * g + 1
            d_step(jo, 1, 0, False, False)
            d_step(jo + 1, 0, 1, False, False)
            return carry

        lax.fori_loop(0, (_CHTV - 2) // 2, d_pair, 0)
        d_step(_CHTV - 1, 1, 0, False, True)
        drain(c_desc(_CHTV - 1, 1))

        plsc.subcore_barrier()
        pltpu.sync_copy(acc_sh.at[pl.ds(s * _NROW, _NROW)],
                        den_out.at[c, pl.ds(s * _NROW, _NROW), :])

    return k(vt, exrep, dst1d, src2, zrows)


def _combine(acc, den, xr, wbeta, bn):
    use_bn = bn is not None

    def body(acc_ref, den_ref, xr_ref, w1_ref, w2_ref, w3_ref, *rest):
        if use_bn:
            g_ref, bb_ref, h_ref = rest
        else:
            (h_ref,) = rest
        a0 = acc_ref[0]
        a1 = acc_ref[1]
        dn = den_ref[0] + den_ref[1]
        o = (a0[:, 0:_D] / (dn[:, 0:1] + 1e-16)
             + a0[:, _D:2 * _D] / (dn[:, 1:2] + 1e-16)
             + a1[:, 0:_D] / (dn[:, 2:3] + 1e-16)
             + a1[:, _D:2 * _D] / (dn[:, 3:4] + 1e-16)) * 0.25
        o = o[0:_N]
        xrv = xr_ref[...]
        beta = jax.nn.sigmoid(
            jnp.dot(o, w1_ref[...], preferred_element_type=_F32)
            + jnp.dot(xrv, w2_ref[...], preferred_element_type=_F32)
            + jnp.dot(o - xrv, w3_ref[...], preferred_element_type=_F32))
        y = beta * xrv + (1.0 - beta) * o
        y = jnp.maximum(y, 0.0)
        if use_bn:
            mean = jnp.mean(y, axis=0, keepdims=True)
            var = jnp.mean((y - mean) * (y - mean), axis=0, keepdims=True)
            y = (y - mean) / jnp.sqrt(var + 1e-5) * g_ref[...] + bb_ref[...]
        h_ref[...] = y

    args = [acc, den, xr, wbeta[0:64], wbeta[64:128], wbeta[128:192]]
    if use_bn:
        args += [bn['gamma'].reshape(1, -1), bn['beta'].reshape(1, -1)]
    return pl.pallas_call(
        body, out_shape=jax.ShapeDtypeStruct((_N, _D), _F32))(*args)


def _s2s_decode(x, batch2d, shear2d, wih_t, whh_t, bsum, winter, binter,
                wmu, bmu, wlv, blv, w1, b1, w2, b2, wa, ba, we, be):
    def body(x_ref, b_ref, shear_ref, wih_ref, whh_ref, bsum_ref,
             winter_ref, binter_ref, wmu_ref, bmu_ref, wlv_ref, blv_ref,
             w1_ref, b1_ref, w2_ref, b2_ref, wa_ref, ba_ref, we_ref, be_ref,
             triu_ref, node_ref, mu_ref, lv_ref):
        xv = x_ref[...]
        bidx = b_ref[...]
        cols = lax.broadcasted_iota(jnp.int32, (1, _B), 1)
        onehot = (bidx == cols).astype(_F32)
        h = jnp.zeros((_B, _EMB), _F32)
        cc = jnp.zeros((_B, _EMB), _F32)
        q_star = jnp.zeros((_B, 2 * _EMB), _F32)
        for _ in range(4):
            gates = (jnp.dot(q_star, wih_ref[...], preferred_element_type=_F32)
                     + jnp.dot(h, whh_ref[...], preferred_element_type=_F32)
                     + bsum_ref[...])
            i_g = jax.nn.sigmoid(gates[:, 0:_EMB])
            f_g = jax.nn.sigmoid(gates[:, _EMB:2 * _EMB])
            g_g = jnp.tanh(gates[:, 2 * _EMB:3 * _EMB])
            o_g = jax.nn.sigmoid(gates[:, 3 * _EMB:4 * _EMB])
            cc = f_g * cc + i_g * g_g
            h = o_g * jnp.tanh(cc)
            hn = jnp.dot(onehot, h, preferred_element_type=_F32)
            e = jnp.sum(xv * hn, axis=1, keepdims=True)
            e_m = jnp.where(onehot > 0.5, e, -1e30)
            m = jnp.max(e_m, axis=0, keepdims=True)
            mb = jnp.sum(onehot * m, axis=1, keepdims=True)
            ex = jnp.exp(e - mb)
            seg = lax.dot_general(onehot, ex, (((0,), (0,)), ((), ())),
                                  preferred_element_type=_F32)
            sb = jnp.dot(onehot, seg, preferred_element_type=_F32)
            a = ex / (sb + 1e-16)
            r = lax.dot_general(onehot * a, xv, (((0,), (0,)), ((), ())),
                                preferred_element_type=_F32)
            q_star = jnp.concatenate([h, r], axis=1)
        t = jnp.dot(q_star, winter_ref[...], preferred_element_type=_F32) + binter_ref[...]
        colmask = (lax.broadcasted_iota(jnp.int32, (1, 128), 1) == 127).astype(_F32)
        t = t + shear_ref[...] * colmask
        mu = jnp.dot(t, wmu_ref[...], preferred_element_type=_F32) + bmu_ref[...]
        lv = jnp.dot(t, wlv_ref[...], preferred_element_type=_F32) + blv_ref[...]
        d = jnp.maximum(jnp.dot(mu, w1_ref[...], preferred_element_type=_F32) + b1_ref[...], 0.0)
        d = jnp.maximum(jnp.dot(d, w2_ref[...], preferred_element_type=_F32) + b2_ref[...], 0.0)
        node_ref[...] = jnp.dot(d, wa_ref[...], preferred_element_type=_F32) + ba_ref[...]
        triu_ref[...] = jnp.dot(d, we_ref[...], preferred_element_type=_F32) + be_ref[...]
        mu_ref[...] = mu
        lv_ref[...] = lv

    return pl.pallas_call(
        body,
        out_shape=[jax.ShapeDtypeStruct((_B, 210), _F32),
                   jax.ShapeDtypeStruct((_B, 1380), _F32),
                   jax.ShapeDtypeStruct((_B, 128), _F32),
                   jax.ShapeDtypeStruct((_B, 128), _F32)],
    )(x, batch2d, shear2d, wih_t, whh_t, bsum, winter, binter,
      wmu, bmu, wlv, blv, w1, b1, w2, b2, wa, ba, we, be)


def kernel(x, edge_index, shear_modulus, batch_index, params):
    src = edge_index[0]
    dst = edge_index[1]
    src2 = jnp.stack([src, src + _N], axis=0).reshape(-1)
    zrows = jnp.zeros((_NROW, _DM), _F32)

    h = x
    for cname, bnname in (('conv1', 'bn1'), ('conv2', 'bn2'),
                          ('conv3', 'bn3'), ('conv4', None)):
        p = params[cname]
        q_t, k_t, vt, xr = _proj(h, p)
        alpha64 = _gather_alpha(q_t, k_t, dst, src)
        alpha, mx = _alpha_red(alpha64)
        exrep = _exrep(alpha, mx)
        acc, den = _edge_scatter(vt.reshape(_NC * _N, _DM), exrep,
                                 dst, src2, zrows)
        bn = params[bnname] if bnname is not None else None
        h = _combine(acc, den, xr, p['Wbeta'], bn)

    lp = params['lstm']
    ip = params['lin_inter']
    winter = jnp.pad(ip['W'], ((0, 0), (0, 1)))
    binter = jnp.pad(ip['b'], (0, 1)).reshape(1, -1)
    triu2, node2, mu, logvar = _s2s_decode(
        h, batch_index.reshape(-1, 1), shear_modulus.reshape(-1, 1),
        lp['Wih'].T, lp['Whh'].T, (lp['bih'] + lp['bhh']).reshape(1, -1),
        winter, binter,
        params['lin_mu']['W'], params['lin_mu']['b'].reshape(1, -1),
        params['lin_logvar']['W'], params['lin_logvar']['b'].reshape(1, -1),
        params['lin1']['W'], params['lin1']['b'].reshape(1, -1),
        params['lin2']['W'], params['lin2']['b'].reshape(1, -1),
        params['atom']['W'], params['atom']['b'].reshape(1, -1),
        params['edge']['W'], params['edge']['b'].reshape(1, -1))
    return triu2.reshape(-1), node2.reshape(-1), mu, logvar


# speedup vs baseline: 1.0303x; 1.0303x over previous
import functools

import jax
import jax.numpy as jnp
from jax import lax
from jax.experimental import pallas as pl
from jax.experimental.pallas import tpu as pltpu
from jax.experimental.pallas import tpu_sc as plsc

_F32 = jnp.float32

_N = 10000
_E = 160000
_H = 4
_D = 64
_HD = _H * _D
_B = 64
_EMB = 64
_DM = 128

_NC, _NS = 2, 16
_NW = _NC * _NS
_EW = _E // _NW
_CG = 40
_NCH = _EW // _CG
_ET = _E // _NS
_CGV = 40
_CHTV = _ET // _CGV
_NP = 10240
_NROW = _NP // _NS


def _mesh():
    return plsc.VectorSubcoreMesh(
        core_axis_name="c", subcore_axis_name="s",
        num_cores=_NC, num_subcores=_NS)


def _proj(x, p):
    n, c = x.shape
    rb = 2000

    def body(x_ref, wq, bq, wk, bk, wv, bv, ws, bs, q_o, k_o, vt_o, xr_o):
        xb = x_ref[...]
        q_o[...] = jnp.dot(xb, wq[...], preferred_element_type=_F32) + bq[...]
        k_o[...] = jnp.dot(xb, wk[...], preferred_element_type=_F32) + bk[...]
        v = jnp.dot(xb, wv[...], preferred_element_type=_F32) + bv[...]
        vt_o[0] = v[:, 0:_DM]
        vt_o[1] = v[:, _DM:2 * _DM]
        xr_o[...] = jnp.dot(xb, ws[...], preferred_element_type=_F32) + bs[...]

    def full(shape):
        return pl.BlockSpec(shape, lambda i: (0, 0))

    row = lambda w: pl.BlockSpec((rb, w), lambda i: (i, 0))
    return pl.pallas_call(
        body, grid=(n // rb,),
        in_specs=[row(c),
                  full((c, _HD)), full((1, _HD)),
                  full((c, _HD)), full((1, _HD)),
                  full((c, _HD)), full((1, _HD)),
                  full((c, _D)), full((1, _D))],
        out_specs=[row(_HD), row(_HD),
                   pl.BlockSpec((_NC, rb, _DM), lambda i: (0, i, 0)), row(_D)],
        out_shape=[jax.ShapeDtypeStruct((n, _HD), _F32)] * 2
                 + [jax.ShapeDtypeStruct((_NC, n, _DM), _F32),
                    jax.ShapeDtypeStruct((n, _D), _F32)],
    )(x, p['Wq'], p['bq'].reshape(1, -1), p['Wk'], p['bk'].reshape(1, -1),
      p['Wv'], p['bv'].reshape(1, -1), p['Wskip'], p['bskip'].reshape(1, -1))


def _gather_qk(q_tab, k_tab, dst3, src3):
    @functools.partial(
        pl.kernel,
        out_type=[jax.ShapeDtypeStruct((_E, _HD), _F32)] * 2,
        mesh=_mesh(),
        scratch_types=[
            pltpu.VMEM((_NCH, _CG), jnp.int32),
            pltpu.VMEM((_NCH, _CG), jnp.int32),
            pltpu.VMEM((2, _CG, _HD), _F32),
            pltpu.VMEM((2, _CG, _HD), _F32),
            pltpu.SemaphoreType.DMA,
            pltpu.SemaphoreType.DMA,
            pltpu.SemaphoreType.DMA,
            pltpu.SemaphoreType.DMA,
        ],
    )
    def k(q_hbm, k_hbm, d_hbm, s_hbm, qd_hbm, ks_hbm,
          dv, sv, qv, kv, gs0, gs1, ss0, ss1):
        wid = lax.axis_index("s") * _NC + lax.axis_index("c")
        ebase = wid * _EW
        pltpu.sync_copy(d_hbm.at[wid], dv)
        pltpu.sync_copy(s_hbm.at[wid], sv)
        gsem = (gs0, gs1)
        ssem = (ss0, ss1)

        def g_desc(j, b):
            return (pltpu.make_async_copy(q_hbm.at[dv.at[j]], qv.at[b], gsem[b]),
                    pltpu.make_async_copy(k_hbm.at[sv.at[j]], kv.at[b], gsem[b]))

        def s_desc(j, b):
            off = ebase + j * _CG
            return (pltpu.make_async_copy(qv.at[b], qd_hbm.at[pl.ds(off, _CG)], ssem[b]),
                    pltpu.make_async_copy(kv.at[b], ks_hbm.at[pl.ds(off, _CG)], ssem[b]))

        def fire(ds):
            for d in ds:
                d.start()

        def drain(ds):
            for d in ds:
                d.wait()

        fire(g_desc(0, 0))
        fire(g_desc(1, 1))
        drain(g_desc(0, 0))
        fire(s_desc(0, 0))

        def pair(g, carry):
            jo = 2 * g + 1
            drain(s_desc(jo - 1, 0))
            fire(g_desc(jo + 1, 0))
            drain(g_desc(jo, 1))
            fire(s_desc(jo, 1))
            je = jo + 1
            drain(s_desc(je - 1, 1))
            fire(g_desc(je + 1, 1))
            drain(g_desc(je, 0))
            fire(s_desc(je, 0))
            return carry

        lax.fori_loop(0, (_NCH - 3) // 2, pair, 0)
        jo = _NCH - 2
        drain(s_desc(jo - 1, 0))
        fire(g_desc(jo + 1, 0))
        drain(g_desc(jo, 1))
        fire(s_desc(jo, 1))
        je = _NCH - 1
        drain(s_desc(je - 1, 1))
        drain(g_desc(je, 0))
        fire(s_desc(je, 0))
        drain(s_desc(je, 0))

    return k(q_tab, k_tab, dst3, src3)


def _alpha_max(qd, ks):
    rb = 4000

    def body(qd_ref, ks_ref, alpha_ref, mx_ref):
        q = qd_ref[...]
        kk = ks_ref[...]
        parts = [jnp.sum(q[:, h * _D:(h + 1) * _D] * kk[:, h * _D:(h + 1) * _D],
                         axis=1, keepdims=True) for h in range(_H)]
        a = jnp.concatenate(parts, axis=1) * 0.125
        alpha_ref[...] = a
        bm = jnp.max(a, axis=0, keepdims=True)
        i = pl.program_id(0)

        @pl.when(i == 0)
        def _():
            mx_ref[...] = bm

        @pl.when(i != 0)
        def _():
            mx_ref[...] = jnp.maximum(mx_ref[...], bm)

    return pl.pallas_call(
        body, grid=(_E // rb,),
        in_specs=[pl.BlockSpec((rb, _HD), lambda i: (i, 0)),
                  pl.BlockSpec((rb, _HD), lambda i: (i, 0))],
        out_specs=[pl.BlockSpec((rb, _H), lambda i: (i, 0)),
                   pl.BlockSpec((1, _H), lambda i: (0, 0))],
        out_shape=[jax.ShapeDtypeStruct((_E, _H), _F32),
                   jax.ShapeDtypeStruct((1, _H), _F32)],
    )(qd, ks)


def _exrep(alpha, mx):
    rb = 8000

    def body(alpha_ref, mx_ref, er_ref):
        ex = jnp.exp(alpha_ref[...] - mx_ref[...])
        for c in range(_NC):
            e0 = jnp.broadcast_to(ex[:, 2 * c:2 * c + 1], (rb, 16))
            e1 = jnp.broadcast_to(ex[:, 2 * c + 1:2 * c + 2], (rb, 16))
            er_ref[c] = jnp.concatenate([e0, e1], axis=1)

    return pl.pallas_call(
        body, grid=(_E // rb,),
        in_specs=[pl.BlockSpec((rb, _H), lambda i: (i, 0)),
                  pl.BlockSpec((1, _H), lambda i: (0, 0))],
        out_specs=pl.BlockSpec((_NC, rb, 32), lambda i: (0, i, 0)),
        out_shape=jax.ShapeDtypeStruct((_NC, _E, 32), _F32),
    )(alpha, mx)


def _edge_scatter(vt, exrep, dst1d, src2, zrows):
    @functools.partial(
        pl.kernel,
        out_type=[jax.ShapeDtypeStruct((_NC, _NP, _DM), _F32)] * 2,
        mesh=_mesh(),
        scratch_types=[
            pltpu.VMEM((2, _CGV), jnp.int32),
            pltpu.VMEM((2, _CGV), jnp.int32),
            pltpu.VMEM((2, _CGV, _DM), _F32),
            pltpu.VMEM((2, _CGV, 32), _F32),
            pltpu.VMEM_SHARED((_NP, _DM), _F32),
            pltpu.SemaphoreType.DMA,
            pltpu.SemaphoreType.DMA,
            pltpu.SemaphoreType.DMA,
            pltpu.SemaphoreType.DMA,
            pltpu.SemaphoreType.DMA,
            pltpu.SemaphoreType.DMA,
        ],
    )
    def k(vt_hbm, er_hbm, d_hbm, s_hbm, z_hbm, acc_out, den_out,
          didx, sidx, vb, exr, acc_sh, is0, is1, gs0, gs1, cs0, cs1):
        c = lax.axis_index("c")
        s = lax.axis_index("s")
        pltpu.sync_copy(z_hbm, acc_sh.at[pl.ds(s * _NROW, _NROW)])
        isem = (is0, is1)
        gsem = (gs0, gs1)
        csem = (cs0, cs1)
        iota16 = lax.iota(jnp.int32, 16)
        plsc.subcore_barrier()

        def i_desc(j, b, with_src):
            off = s * _ET + j * _CGV
            ds_ = [pltpu.make_async_copy(
                       d_hbm.at[pl.ds(off, _CGV)], didx.at[b], isem[b]),
                   pltpu.make_async_copy(
                       er_hbm.at[c, pl.ds(off, _CGV), :], exr.at[b], isem[b])]
            if with_src:
                ds_.append(pltpu.make_async_copy(
                    s_hbm.at[pl.ds(c * _E + off, _CGV)], sidx.at[b], isem[b]))
            return ds_

        def g_desc(j, b):
            return (pltpu.make_async_copy(
                vt_hbm.at[sidx.at[b]], vb.at[b], gsem[b]),)

        def c_desc(j, b):
            return (pltpu.make_async_copy(
                vb.at[b], acc_sh.at[didx.at[b]], csem[b]),)

        def fire(ds):
            for d in ds:
                d.start()

        def drain(ds):
            for d in ds:
                d.wait()

        def fire_add(j, b):
            pltpu.async_copy(vb.at[b], acc_sh.at[didx.at[b]], csem[b], add=True)

        def v_compute(b):
            def edge_body(eo, carry):
                for u in range(4):
                    e = eo * 4 + u
                    b0 = exr[b, e, pl.ds(0, 16)]
                    b1 = exr[b, e, pl.ds(16, 16)]
                    for i in range(8):
                        sl = pl.ds(i * 16, 16)
                        bb = b0 if i < 4 else b1
                        vb[b, e, sl] = vb[b, e, sl] * bb
                return carry

            lax.fori_loop(0, _CGV // 4, edge_body, 0)

        def v_step(j, b, nb, first, last):
            drain(g_desc(j, b))
            if not first:
                drain(c_desc(j - 1, nb))
            if not last:
                fire(i_desc(j + 1, nb, True))
            v_compute(b)
            if not last:
                drain(i_desc(j + 1, nb, True))
                fire(g_desc(j + 1, nb))
            fire_add(j, b)

        fire(i_desc(0, 0, True))
        drain(i_desc(0, 0, True))
        fire(g_desc(0, 0))
        v_step(0, 0, 1, True, False)

        def v_pair(g, carry):
            jo = 2 * g + 1
            v_step(jo, 1, 0, False, False)
            v_step(jo + 1, 0, 1, False, False)
            return carry

        lax.fori_loop(0, (_CHTV - 2) // 2, v_pair, 0)
        v_step(_CHTV - 1, 1, 0, False, True)
        drain(c_desc(_CHTV - 1, 1))

        plsc.subcore_barrier()
        pltpu.sync_copy(acc_sh.at[pl.ds(s * _NROW, _NROW)],
                        acc_out.at[c, pl.ds(s * _NROW, _NROW), :])
        plsc.subcore_barrier()
        pltpu.sync_copy(z_hbm, acc_sh.at[pl.ds(s * _NROW, _NROW)])
        plsc.subcore_barrier()

        pltpu.sync_copy(z_hbm.at[pl.ds(0, _CGV)], vb.at[0])
        pltpu.sync_copy(z_hbm.at[pl.ds(0, _CGV)], vb.at[1])

        def d_compute(b):
            def edge_body(eo, carry):
                for u in range(4):
                    e = eo * 4 + u
                    v0 = exr[b, e, pl.ds(0, 16)]
                    v1 = exr[b, e, pl.ds(16, 16)]
                    m = jnp.where(iota16 == 2 * c, v0,
                                  jnp.where(iota16 == 2 * c + 1, v1,
                                            jnp.zeros((16,), _F32)))
                    vb[b, e, pl.ds(0, 16)] = m
                return carry

            lax.fori_loop(0, _CGV // 4, edge_body, 0)

        def d_step(j, b, nb, first, last):
            drain(i_desc(j, b, False))
            if not first:
                drain(c_desc(j - 1, nb))
            if not last:
                fire(i_desc(j + 1, nb, False))
            d_compute(b)
            fire_add(j, b)

        fire(i_desc(0, 0, False))
        d_step(0, 0, 1, True, False)

        def d_pair(g, carry):
            jo = 2 * g + 1
            d_step(jo, 1, 0, False, False)
            d_step(jo + 1, 0, 1, False, False)
            return carry

        lax.fori_loop(0, (_CHTV - 2) // 2, d_pair, 0)
        d_step(_CHTV - 1, 1, 0, False, True)
        drain(c_desc(_CHTV - 1, 1))

        plsc.subcore_barrier()
        pltpu.sync_copy(acc_sh.at[pl.ds(s * _NROW, _NROW)],
                        den_out.at[c, pl.ds(s * _NROW, _NROW), :])

    return k(vt, exrep, dst1d, src2, zrows)


def _combine(acc, den, xr, wbeta, bn):
    use_bn = bn is not None

    def body(acc_ref, den_ref, xr_ref, w1_ref, w2_ref, w3_ref, *rest):
        if use_bn:
            g_ref, bb_ref, h_ref = rest
        else:
            (h_ref,) = rest
        a0 = acc_ref[0]
        a1 = acc_ref[1]
        dn = den_ref[0] + den_ref[1]
        o = (a0[:, 0:_D] / (dn[:, 0:1] + 1e-16)
             + a0[:, _D:2 * _D] / (dn[:, 1:2] + 1e-16)
             + a1[:, 0:_D] / (dn[:, 2:3] + 1e-16)
             + a1[:, _D:2 * _D] / (dn[:, 3:4] + 1e-16)) * 0.25
        o = o[0:_N]
        xrv = xr_ref[...]
        beta = jax.nn.sigmoid(
            jnp.dot(o, w1_ref[...], preferred_element_type=_F32)
            + jnp.dot(xrv, w2_ref[...], preferred_element_type=_F32)
            + jnp.dot(o - xrv, w3_ref[...], preferred_element_type=_F32))
        y = beta * xrv + (1.0 - beta) * o
        y = jnp.maximum(y, 0.0)
        if use_bn:
            mean = jnp.mean(y, axis=0, keepdims=True)
            var = jnp.mean((y - mean) * (y - mean), axis=0, keepdims=True)
            y = (y - mean) / jnp.sqrt(var + 1e-5) * g_ref[...] + bb_ref[...]
        h_ref[...] = y

    args = [acc, den, xr, wbeta[0:64], wbeta[64:128], wbeta[128:192]]
    if use_bn:
        args += [bn['gamma'].reshape(1, -1), bn['beta'].reshape(1, -1)]
    return pl.pallas_call(
        body, out_shape=jax.ShapeDtypeStruct((_N, _D), _F32))(*args)


def _s2s_decode(x, batch2d, shear2d, wih_t, whh_t, bsum, winter, binter,
                wmu, bmu, wlv, blv, w1, b1, w2, b2, wa, ba, we, be):
    def body(x_ref, b_ref, shear_ref, wih_ref, whh_ref, bsum_ref,
             winter_ref, binter_ref, wmu_ref, bmu_ref, wlv_ref, blv_ref,
             w1_ref, b1_ref, w2_ref, b2_ref, wa_ref, ba_ref, we_ref, be_ref,
             triu_ref, node_ref, mu_ref, lv_ref):
        xv = x_ref[...]
        bidx = b_ref[...]
        cols = lax.broadcasted_iota(jnp.int32, (1, _B), 1)
        onehot = (bidx == cols).astype(_F32)
        h = jnp.zeros((_B, _EMB), _F32)
        cc = jnp.zeros((_B, _EMB), _F32)
        q_star = jnp.zeros((_B, 2 * _EMB), _F32)
        for _ in range(4):
            gates = (jnp.dot(q_star, wih_ref[...], preferred_element_type=_F32)
                     + jnp.dot(h, whh_ref[...], preferred_element_type=_F32)
                     + bsum_ref[...])
            i_g = jax.nn.sigmoid(gates[:, 0:_EMB])
            f_g = jax.nn.sigmoid(gates[:, _EMB:2 * _EMB])
            g_g = jnp.tanh(gates[:, 2 * _EMB:3 * _EMB])
            o_g = jax.nn.sigmoid(gates[:, 3 * _EMB:4 * _EMB])
            cc = f_g * cc + i_g * g_g
            h = o_g * jnp.tanh(cc)
            hn = jnp.dot(onehot, h, preferred_element_type=_F32)
            e = jnp.sum(xv * hn, axis=1, keepdims=True)
            e_m = jnp.where(onehot > 0.5, e, -1e30)
            m = jnp.max(e_m, axis=0, keepdims=True)
            mb = jnp.sum(onehot * m, axis=1, keepdims=True)
            ex = jnp.exp(e - mb)
            seg = lax.dot_general(onehot, ex, (((0,), (0,)), ((), ())),
                                  preferred_element_type=_F32)
            sb = jnp.dot(onehot, seg, preferred_element_type=_F32)
            a = ex / (sb + 1e-16)
            r = lax.dot_general(onehot * a, xv, (((0,), (0,)), ((), ())),
                                preferred_element_type=_F32)
            q_star = jnp.concatenate([h, r], axis=1)
        t = jnp.dot(q_star, winter_ref[...], preferred_element_type=_F32) + binter_ref[...]
        colmask = (lax.broadcasted_iota(jnp.int32, (1, 128), 1) == 127).astype(_F32)
        t = t + shear_ref[...] * colmask
        mu = jnp.dot(t, wmu_ref[...], preferred_element_type=_F32) + bmu_ref[...]
        lv = jnp.dot(t, wlv_ref[...], preferred_element_type=_F32) + blv_ref[...]
        d = jnp.maximum(jnp.dot(mu, w1_ref[...], preferred_element_type=_F32) + b1_ref[...], 0.0)
        d = jnp.maximum(jnp.dot(d, w2_ref[...], preferred_element_type=_F32) + b2_ref[...], 0.0)
        node_ref[...] = jnp.dot(d, wa_ref[...], preferred_element_type=_F32) + ba_ref[...]
        triu_ref[...] = jnp.dot(d, we_ref[...], preferred_element_type=_F32) + be_ref[...]
        mu_ref[...] = mu
        lv_ref[...] = lv

    return pl.pallas_call(
        body,
        out_shape=[jax.ShapeDtypeStruct((_B, 210), _F32),
                   jax.ShapeDtypeStruct((_B, 1380), _F32),
                   jax.ShapeDtypeStruct((_B, 128), _F32),
                   jax.ShapeDtypeStruct((_B, 128), _F32)],
    )(x, batch2d, shear2d, wih_t, whh_t, bsum, winter, binter,
      wmu, bmu, wlv, blv, w1, b1, w2, b2, wa, ba, we, be)


def kernel(x, edge_index, shear_modulus, batch_index, params):
    src = edge_index[0]
    dst = edge_index[1]
    src2 = jnp.stack([src, src + _N], axis=0).reshape(-1)
    dst_g = dst.reshape(_NW, _NCH, _CG)
    src_g = src.reshape(_NW, _NCH, _CG)
    zrows = jnp.zeros((_NROW, _DM), _F32)

    h = x
    for cname, bnname in (('conv1', 'bn1'), ('conv2', 'bn2'),
                          ('conv3', 'bn3'), ('conv4', None)):
        p = params[cname]
        q_t, k_t, vt, xr = _proj(h, p)
        qd, ks = _gather_qk(q_t, k_t, dst_g, src_g)
        alpha, mx = _alpha_max(qd, ks)
        exrep = _exrep(alpha, mx)
        acc, den = _edge_scatter(vt.reshape(_NC * _N, _DM), exrep,
                                 dst, src2, zrows)
        bn = params[bnname] if bnname is not None else None
        h = _combine(acc, den, xr, p['Wbeta'], bn)

    lp = params['lstm']
    ip = params['lin_inter']
    winter = jnp.pad(ip['W'], ((0, 0), (0, 1)))
    binter = jnp.pad(ip['b'], (0, 1)).reshape(1, -1)
    triu2, node2, mu, logvar = _s2s_decode(
        h, batch_index.reshape(-1, 1), shear_modulus.reshape(-1, 1),
        lp['Wih'].T, lp['Whh'].T, (lp['bih'] + lp['bhh']).reshape(1, -1),
        winter, binter,
        params['lin_mu']['W'], params['lin_mu']['b'].reshape(1, -1),
        params['lin_logvar']['W'], params['lin_logvar']['b'].reshape(1, -1),
        params['lin1']['W'], params['lin1']['b'].reshape(1, -1),
        params['lin2']['W'], params['lin2']['b'].reshape(1, -1),
        params['atom']['W'], params['atom']['b'].reshape(1, -1),
        params['edge']['W'], params['edge']['b'].reshape(1, -1))
    return triu2.reshape(-1), node2.reshape(-1), mu, logvar


# speedup vs baseline: 1.3090x; 1.2705x over previous
import functools

import jax
import jax.numpy as jnp
from jax import lax
from jax.experimental import pallas as pl
from jax.experimental.pallas import tpu as pltpu
from jax.experimental.pallas import tpu_sc as plsc

_F32 = jnp.float32

_N = 10000
_E = 160000
_H = 4
_D = 64
_HD = _H * _D
_B = 64
_EMB = 64
_DM = 128

_NC, _NS = 2, 16
_NW = _NC * _NS
_EW = _E // _NW
_CG = 40
_NCH = _EW // _CG
_ET = _E // _NS
_CGV = 80
_CHTV = _ET // _CGV
_NP = 10240
_NROW = _NP // _NS


def _mesh():
    return plsc.VectorSubcoreMesh(
        core_axis_name="c", subcore_axis_name="s",
        num_cores=_NC, num_subcores=_NS)


def _proj(x, p):
    n, c = x.shape
    rb = 2000

    def body(x_ref, wq, bq, wk, bk, wv, bv, ws, bs, q_o, k_o, v_o, xr_o):
        xb = x_ref[...]
        q_o[...] = jnp.dot(xb, wq[...], preferred_element_type=_F32) + bq[...]
        k_o[...] = jnp.dot(xb, wk[...], preferred_element_type=_F32) + bk[...]
        v_o[...] = jnp.dot(xb, wv[...], preferred_element_type=_F32) + bv[...]
        xr_o[...] = jnp.dot(xb, ws[...], preferred_element_type=_F32) + bs[...]

    def full(shape):
        return pl.BlockSpec(shape, lambda i: (0, 0))

    row = lambda w: pl.BlockSpec((rb, w), lambda i: (i, 0))
    return pl.pallas_call(
        body, grid=(n // rb,),
        in_specs=[row(c),
                  full((c, _HD)), full((1, _HD)),
                  full((c, _HD)), full((1, _HD)),
                  full((c, _HD)), full((1, _HD)),
                  full((c, _D)), full((1, _D))],
        out_specs=[row(_HD), row(_HD), row(_HD), row(_D)],
        out_shape=[jax.ShapeDtypeStruct((n, _HD), _F32)] * 3
                 + [jax.ShapeDtypeStruct((n, _D), _F32)],
    )(x, p['Wq'], p['bq'].reshape(1, -1), p['Wk'], p['bk'].reshape(1, -1),
      p['Wv'], p['bv'].reshape(1, -1), p['Wskip'], p['bskip'].reshape(1, -1))


def _gather_edges(q_tab, k_tab, v_tab, dst3, src3):
    @functools.partial(
        pl.kernel,
        out_type=[jax.ShapeDtypeStruct((_E, _HD), _F32)] * 3,
        mesh=_mesh(),
        scratch_types=[
            pltpu.VMEM((_NCH, _CG), jnp.int32),
            pltpu.VMEM((_NCH, _CG), jnp.int32),
            pltpu.VMEM((2, _CG, _HD), _F32),
            pltpu.VMEM((2, _CG, _HD), _F32),
            pltpu.VMEM((2, _CG, _HD), _F32),
            pltpu.SemaphoreType.DMA,
            pltpu.SemaphoreType.DMA,
            pltpu.SemaphoreType.DMA,
            pltpu.SemaphoreType.DMA,
        ],
    )
    def k(q_hbm, k_hbm, v_hbm, d_hbm, s_hbm, qd_hbm, ks_hbm, vs_hbm,
          dv, sv, qv, kv, vv, gs0, gs1, ss0, ss1):
        wid = lax.axis_index("s") * _NC + lax.axis_index("c")
        ebase = wid * _EW
        pltpu.sync_copy(d_hbm.at[wid], dv)
        pltpu.sync_copy(s_hbm.at[wid], sv)
        gsem = (gs0, gs1)
        ssem = (ss0, ss1)

        def g_desc(j, b):
            return (pltpu.make_async_copy(q_hbm.at[dv.at[j]], qv.at[b], gsem[b]),
                    pltpu.make_async_copy(k_hbm.at[sv.at[j]], kv.at[b], gsem[b]),
                    pltpu.make_async_copy(v_hbm.at[sv.at[j]], vv.at[b], gsem[b]))

        def s_desc(j, b):
            off = ebase + j * _CG
            return (pltpu.make_async_copy(qv.at[b], qd_hbm.at[pl.ds(off, _CG)], ssem[b]),
                    pltpu.make_async_copy(kv.at[b], ks_hbm.at[pl.ds(off, _CG)], ssem[b]),
                    pltpu.make_async_copy(vv.at[b], vs_hbm.at[pl.ds(off, _CG)], ssem[b]))

        def fire(ds):
            for d in ds:
                d.start()

        def drain(ds):
            for d in ds:
                d.wait()

        fire(g_desc(0, 0))
        fire(g_desc(1, 1))
        drain(g_desc(0, 0))
        fire(s_desc(0, 0))

        def pair(g, carry):
            jo = 2 * g + 1
            drain(s_desc(jo - 1, 0))
            fire(g_desc(jo + 1, 0))
            drain(g_desc(jo, 1))
            fire(s_desc(jo, 1))
            je = jo + 1
            drain(s_desc(je - 1, 1))
            fire(g_desc(je + 1, 1))
            drain(g_desc(je, 0))
            fire(s_desc(je, 0))
            return carry

        lax.fori_loop(0, (_NCH - 3) // 2, pair, 0)
        jo = _NCH - 2
        drain(s_desc(jo - 1, 0))
        fire(g_desc(jo + 1, 0))
        drain(g_desc(jo, 1))
        fire(s_desc(jo, 1))
        je = _NCH - 1
        drain(s_desc(je - 1, 1))
        drain(g_desc(je, 0))
        fire(s_desc(je, 0))
        drain(s_desc(je, 0))

    return k(q_tab, k_tab, v_tab, dst3, src3)


def _alpha_max(qd, ks):
    rb = 4000

    def body(qd_ref, ks_ref, alpha_ref, mx_ref):
        q = qd_ref[...]
        kk = ks_ref[...]
        parts = [jnp.sum(q[:, h * _D:(h + 1) * _D] * kk[:, h * _D:(h + 1) * _D],
                         axis=1, keepdims=True) for h in range(_H)]
        a = jnp.concatenate(parts, axis=1) * 0.125
        alpha_ref[...] = a
        bm = jnp.max(a, axis=0, keepdims=True)
        i = pl.program_id(0)

        @pl.when(i == 0)
        def _():
            mx_ref[...] = bm

        @pl.when(i != 0)
        def _():
            mx_ref[...] = jnp.maximum(mx_ref[...], bm)

    return pl.pallas_call(
        body, grid=(_E // rb,),
        in_specs=[pl.BlockSpec((rb, _HD), lambda i: (i, 0)),
                  pl.BlockSpec((rb, _HD), lambda i: (i, 0))],
        out_specs=[pl.BlockSpec((rb, _H), lambda i: (i, 0)),
                   pl.BlockSpec((1, _H), lambda i: (0, 0))],
        out_shape=[jax.ShapeDtypeStruct((_E, _H), _F32),
                   jax.ShapeDtypeStruct((1, _H), _F32)],
    )(qd, ks)


def _build_msg(alpha, vs, mx):
    rb = 4000

    def body(alpha_ref, vs_ref, mx_ref, msg_ref, den_ref):
        ex = jnp.exp(alpha_ref[...] - mx_ref[...])
        v = vs_ref[...]
        for c in range(_NC):
            h0, h1 = 2 * c, 2 * c + 1
            m0 = v[:, h0 * _D:(h0 + 1) * _D] * ex[:, h0:h0 + 1]
            m1 = v[:, h1 * _D:(h1 + 1) * _D] * ex[:, h1:h1 + 1]
            msg_ref[c] = jnp.concatenate([m0, m1], axis=1)
        z = jnp.zeros((rb, _DM - _H), _F32)
        den_ref[...] = jnp.concatenate([ex, z], axis=1)

    return pl.pallas_call(
        body, grid=(_E // rb,),
        in_specs=[pl.BlockSpec((rb, _H), lambda i: (i, 0)),
                  pl.BlockSpec((rb, _HD), lambda i: (i, 0)),
                  pl.BlockSpec((1, _H), lambda i: (0, 0))],
        out_specs=[pl.BlockSpec((_NC, rb, _DM), lambda i: (0, i, 0)),
                   pl.BlockSpec((rb, _DM), lambda i: (i, 0))],
        out_shape=[jax.ShapeDtypeStruct((_NC, _E, _DM), _F32),
                   jax.ShapeDtypeStruct((_E, _DM), _F32)],
    )(alpha, vs, mx)


def _scatter_msg(msg, den_msg, dst_v3, zrows):
    @functools.partial(
        pl.kernel,
        out_type=[jax.ShapeDtypeStruct((_NC, _NP, _DM), _F32)] * 2,
        mesh=_mesh(),
        scratch_types=[
            pltpu.VMEM((_CHTV, _CGV), jnp.int32),
            pltpu.VMEM((2, _CGV, _DM), _F32),
            pltpu.VMEM_SHARED((_NP, _DM), _F32),
            pltpu.SemaphoreType.DMA,
            pltpu.SemaphoreType.DMA,
            pltpu.SemaphoreType.DMA,
            pltpu.SemaphoreType.DMA,
        ],
    )
    def k(msg_hbm, den_hbm, dv_hbm, z_hbm, acc_out, den_out,
          dv, mv, acc_sh, ls0, ls1, cs0, cs1):
        c = lax.axis_index("c")
        s = lax.axis_index("s")
        pltpu.sync_copy(z_hbm, acc_sh.at[pl.ds(s * _NROW, _NROW)])
        pltpu.sync_copy(dv_hbm.at[s], dv)
        lsem = (ls0, ls1)
        csem = (cs0, cs1)
        plsc.subcore_barrier()

        def pipe_pass(src_hbm, base3):
            def l_desc(j, b):
                return (pltpu.make_async_copy(
                    src_hbm.at[pl.ds(base3(j), _CGV), :], mv.at[b], lsem[b]),)

            def c_desc(j, b):
                return (pltpu.make_async_copy(
                    mv.at[b], acc_sh.at[dv.at[j]], csem[b]),)

            def fire(ds, add=False):
                for d in ds:
                    d.start()

            def drain(ds):
                for d in ds:
                    d.wait()

            def fire_add(j, b):
                pltpu.async_copy(mv.at[b], acc_sh.at[dv.at[j]], csem[b], add=True)

            fire(l_desc(0, 0))
            fire(l_desc(1, 1))
            drain(l_desc(0, 0))
            fire_add(0, 0)

            def pair(g, carry):
                jo = 2 * g + 1
                drain(c_desc(jo - 1, 0))
                fire(l_desc(jo + 1, 0))
                drain(l_desc(jo, 1))
                fire_add(jo, 1)
                je = jo + 1
                drain(c_desc(je - 1, 1))
                fire(l_desc(je + 1, 1))
                drain(l_desc(je, 0))
                fire_add(je, 0)
                return carry

            lax.fori_loop(0, (_CHTV - 3) // 2, pair, 0)
            jo = _CHTV - 2
            drain(c_desc(jo - 1, 0))
            fire(l_desc(jo + 1, 0))
            drain(l_desc(jo, 1))
            fire_add(jo, 1)
            je = _CHTV - 1
            drain(c_desc(je - 1, 1))
            drain(l_desc(je, 0))
            fire_add(je, 0)
            drain(c_desc(je, 0))

        pipe_pass(msg_hbm.at[c], lambda j: s * _ET + j * _CGV)
        plsc.subcore_barrier()
        pltpu.sync_copy(acc_sh.at[pl.ds(s * _NROW, _NROW)],
                        acc_out.at[c, pl.ds(s * _NROW, _NROW), :])
        plsc.subcore_barrier()
        pltpu.sync_copy(z_hbm, acc_sh.at[pl.ds(s * _NROW, _NROW)])
        plsc.subcore_barrier()

        @pl.when(s % _NC == c)
        def _():
            pipe_pass(den_hbm, lambda j: s * _ET + j * _CGV)

        plsc.subcore_barrier()
        pltpu.sync_copy(acc_sh.at[pl.ds(s * _NROW, _NROW)],
                        den_out.at[c, pl.ds(s * _NROW, _NROW), :])

    return k(msg, den_msg, dst_v3, zrows)


def _combine(acc, den, xr, wbeta, bn):
    use_bn = bn is not None

    def body(acc_ref, den_ref, xr_ref, w1_ref, w2_ref, w3_ref, *rest):
        if use_bn:
            g_ref, bb_ref, h_ref = rest
        else:
            (h_ref,) = rest
        a0 = acc_ref[0]
        a1 = acc_ref[1]
        dn = den_ref[0] + den_ref[1]
        o = (a0[:, 0:_D] / (dn[:, 0:1] + 1e-16)
             + a0[:, _D:2 * _D] / (dn[:, 1:2] + 1e-16)
             + a1[:, 0:_D] / (dn[:, 2:3] + 1e-16)
             + a1[:, _D:2 * _D] / (dn[:, 3:4] + 1e-16)) * 0.25
        o = o[0:_N]
        xrv = xr_ref[...]
        beta = jax.nn.sigmoid(
            jnp.dot(o, w1_ref[...], preferred_element_type=_F32)
            + jnp.dot(xrv, w2_ref[...], preferred_element_type=_F32)
            + jnp.dot(o - xrv, w3_ref[...], preferred_element_type=_F32))
        y = beta * xrv + (1.0 - beta) * o
        y = jnp.maximum(y, 0.0)
        if use_bn:
            mean = jnp.mean(y, axis=0, keepdims=True)
            var = jnp.mean((y - mean) * (y - mean), axis=0, keepdims=True)
            y = (y - mean) / jnp.sqrt(var + 1e-5) * g_ref[...] + bb_ref[...]
        h_ref[...] = y

    args = [acc, den, xr, wbeta[0:64], wbeta[64:128], wbeta[128:192]]
    if use_bn:
        args += [bn['gamma'].reshape(1, -1), bn['beta'].reshape(1, -1)]
    return pl.pallas_call(
        body, out_shape=jax.ShapeDtypeStruct((_N, _D), _F32))(*args)


def _s2s_decode(x, batch2d, shear2d, wih_t, whh_t, bsum, winter, binter,
                wmu, bmu, wlv, blv, w1, b1, w2, b2, wa, ba, we, be):
    def body(x_ref, b_ref, shear_ref, wih_ref, whh_ref, bsum_ref,
             winter_ref, binter_ref, wmu_ref, bmu_ref, wlv_ref, blv_ref,
             w1_ref, b1_ref, w2_ref, b2_ref, wa_ref, ba_ref, we_ref, be_ref,
             triu_ref, node_ref, mu_ref, lv_ref):
        xv = x_ref[...]
        bidx = b_ref[...]
        cols = lax.broadcasted_iota(jnp.int32, (1, _B), 1)
        onehot = (bidx == cols).astype(_F32)
        h = jnp.zeros((_B, _EMB), _F32)
        cc = jnp.zeros((_B, _EMB), _F32)
        q_star = jnp.zeros((_B, 2 * _EMB), _F32)
        for _ in range(4):
            gates = (jnp.dot(q_star, wih_ref[...], preferred_element_type=_F32)
                     + jnp.dot(h, whh_ref[...], preferred_element_type=_F32)
                     + bsum_ref[...])
            i_g = jax.nn.sigmoid(gates[:, 0:_EMB])
            f_g = jax.nn.sigmoid(gates[:, _EMB:2 * _EMB])
            g_g = jnp.tanh(gates[:, 2 * _EMB:3 * _EMB])
            o_g = jax.nn.sigmoid(gates[:, 3 * _EMB:4 * _EMB])
            cc = f_g * cc + i_g * g_g
            h = o_g * jnp.tanh(cc)
            hn = jnp.dot(onehot, h, preferred_element_type=_F32)
            e = jnp.sum(xv * hn, axis=1, keepdims=True)
            e_m = jnp.where(onehot > 0.5, e, -1e30)
            m = jnp.max(e_m, axis=0, keepdims=True)
            mb = jnp.sum(onehot * m, axis=1, keepdims=True)
            ex = jnp.exp(e - mb)
            seg = lax.dot_general(onehot, ex, (((0,), (0,)), ((), ())),
                                  preferred_element_type=_F32)
            sb = jnp.dot(onehot, seg, preferred_element_type=_F32)
            a = ex / (sb + 1e-16)
            r = lax.dot_general(onehot * a, xv, (((0,), (0,)), ((), ())),
                                preferred_element_type=_F32)
            q_star = jnp.concatenate([h, r], axis=1)
        t = jnp.dot(q_star, winter_ref[...], preferred_element_type=_F32) + binter_ref[...]
        colmask = (lax.broadcasted_iota(jnp.int32, (1, 128), 1) == 127).astype(_F32)
        t = t + shear_ref[...] * colmask
        mu = jnp.dot(t, wmu_ref[...], preferred_element_type=_F32) + bmu_ref[...]
        lv = jnp.dot(t, wlv_ref[...], preferred_element_type=_F32) + blv_ref[...]
        d = jnp.maximum(jnp.dot(mu, w1_ref[...], preferred_element_type=_F32) + b1_ref[...], 0.0)
        d = jnp.maximum(jnp.dot(d, w2_ref[...], preferred_element_type=_F32) + b2_ref[...], 0.0)
        node_ref[...] = jnp.dot(d, wa_ref[...], preferred_element_type=_F32) + ba_ref[...]
        triu_ref[...] = jnp.dot(d, we_ref[...], preferred_element_type=_F32) + be_ref[...]
        mu_ref[...] = mu
        lv_ref[...] = lv

    return pl.pallas_call(
        body,
        out_shape=[jax.ShapeDtypeStruct((_B, 210), _F32),
                   jax.ShapeDtypeStruct((_B, 1380), _F32),
                   jax.ShapeDtypeStruct((_B, 128), _F32),
                   jax.ShapeDtypeStruct((_B, 128), _F32)],
    )(x, batch2d, shear2d, wih_t, whh_t, bsum, winter, binter,
      wmu, bmu, wlv, blv, w1, b1, w2, b2, wa, ba, we, be)


def kernel(x, edge_index, shear_modulus, batch_index, params):
    src = edge_index[0]
    dst = edge_index[1]
    dst_g = dst.reshape(_NW, _NCH, _CG)
    src_g = src.reshape(_NW, _NCH, _CG)
    dst_v3 = dst.reshape(_NS, _CHTV, _CGV)
    zrows = jnp.zeros((_NROW, _DM), _F32)

    h = x
    for cname, bnname in (('conv1', 'bn1'), ('conv2', 'bn2'),
                          ('conv3', 'bn3'), ('conv4', None)):
        p = params[cname]
        q_t, k_t, v_t, xr = _proj(h, p)
        qd, ks, vs = _gather_edges(q_t, k_t, v_t, dst_g, src_g)
        alpha, mx = _alpha_max(qd, ks)
        msg, den_msg = _build_msg(alpha, vs, mx)
        acc, den = _scatter_msg(msg, den_msg, dst_v3, zrows)
        bn = params[bnname] if bnname is not None else None
        h = _combine(acc, den, xr, p['Wbeta'], bn)

    lp = params['lstm']
    ip = params['lin_inter']
    winter = jnp.pad(ip['W'], ((0, 0), (0, 1)))
    binter = jnp.pad(ip['b'], (0, 1)).reshape(1, -1)
    triu2, node2, mu, logvar = _s2s_decode(
        h, batch_index.reshape(-1, 1), shear_modulus.reshape(-1, 1),
        lp['Wih'].T, lp['Whh'].T, (lp['bih'] + lp['bhh']).reshape(1, -1),
        winter, binter,
        params['lin_mu']['W'], params['lin_mu']['b'].reshape(1, -1),
        params['lin_logvar']['W'], params['lin_logvar']['b'].reshape(1, -1),
        params['lin1']['W'], params['lin1']['b'].reshape(1, -1),
        params['lin2']['W'], params['lin2']['b'].reshape(1, -1),
        params['atom']['W'], params['atom']['b'].reshape(1, -1),
        params['edge']['W'], params['edge']['b'].reshape(1, -1))
    return triu2.reshape(-1), node2.reshape(-1), mu, logvar
